# baseline (speedup 1.0000x reference)
import jax
import jax.numpy as jnp
from jax import lax
from jax.experimental import pallas as pl
from jax.experimental.pallas import tpu as pltpu
from jax.experimental.pallas import tpu_sc as plsc

B, S, D = 1024, 200, 128
H = D // 2
N = B * S
NC, NS = 2, 16
NW = NC * NS
PER_W = N // NW
STEP = 128
N_STEPS = PER_W // STEP
NBUF = 3
L = 16
V1, V2 = 48, 256
TBL = (V1 + V2) * H
BLK = STEP * D


def _pe_body(x_hbm, f0_hbm, f1_hbm, pe_hbm, out_hbm,
             fb0, fb1, pe_v, buf0, buf1, buf2,
             six0, six1, six2, so0, so1, so2):
    bufs = (buf0, buf1, buf2)
    six = (six0, six1, six2)
    so = (so0, so1, so2)

    wid = lax.axis_index("s") * NC + lax.axis_index("c")
    base = wid * PER_W

    pltpu.sync_copy(f0_hbm.at[wid], fb0)
    pltpu.sync_copy(f1_hbm.at[wid], fb1)
    pltpu.sync_copy(pe_hbm, pe_v)

    lane = lax.iota(jnp.int32, L)
    lane_j = [jnp.int32(j * L) + lane for j in range(H // L)]
    _gdims = lax.GatherDimensionNumbers(
        offset_dims=(), collapsed_slice_dims=(0,), start_index_map=(0,))

    def bcast(vec, l):
        sel = jnp.full((L, 1), l, jnp.int32)
        return lax.gather(vec, sel, _gdims, (1,),
                          mode=lax.GatherScatterMode.PROMISE_IN_BOUNDS)

    def issue_in(g, b):
        w0 = (base + g * STEP) * D
        pltpu.async_copy(x_hbm.at[pl.ds(w0, BLK)], bufs[b], six[b])

    def wait_in(b):
        pltpu.make_async_copy(x_hbm.at[pl.ds(0, BLK)], bufs[b], six[b]).wait()

    def issue_out(g, b):
        w0 = (base + g * STEP) * D
        pltpu.async_copy(bufs[b], out_hbm.at[pl.ds(w0, BLK)], so[b])

    def wait_out(b):
        pltpu.make_async_copy(bufs[b], out_hbm.at[pl.ds(0, BLK)], so[b]).wait()

    def compute(g, b):
        buf = bufs[b]
        grow = g * STEP

        @plsc.parallel_loop(0, STEP // L, 1, unroll=4)
        def group(tg):
            t0 = tg * L
            bases0 = fb0[pl.ds(grow + t0, L)]
            bases1 = fb1[pl.ds(grow + t0, L)]
            tflat = t0 * D
            for l in range(L):
                b0 = bcast(bases0, l)
                b1 = bcast(bases1, l)
                tb = tflat + (l * D)
                vals = ([plsc.load_gather(pe_v, [b0 + lane_j[j]])
                         for j in range(H // L)] +
                        [plsc.load_gather(pe_v, [b1 + lane_j[j]])
                         for j in range(H // L)])
                for j in range(D // L):
                    plsc.addupdate(buf.at[pl.ds(tb + j * L, L)], vals[j])

    issue_in(0, 0)

    def stage_b(g, c):
        def mk(b):
            def f(_):
                nb = (b + 1) % NBUF

                def issue_next(_):
                    @pl.when(g >= NBUF - 1)
                    def _():
                        wait_out(nb)
                    issue_in(g + 1, nb)
                    return 0

                lax.cond(g + 1 < N_STEPS, issue_next, lambda _: 0, 0)
                wait_in(b)
                compute(g, b)
                issue_out(g, b)
                return 0
            return f

        lax.switch(lax.rem(g, NBUF), [mk(0), mk(1), mk(2)], 0)
        return c

    lax.fori_loop(0, N_STEPS, stage_b, 0)
    for g in range(N_STEPS - NBUF, N_STEPS):
        wait_out(g % NBUF)


@jax.jit
def kernel(inputs, pos, pe1, pe2):
    x = inputs.reshape(N * D)
    p = pos.astype(jnp.int32)
    f0 = (p[..., 0] * H).reshape(NW, PER_W)
    f1 = (p[..., 1] * H + V1 * H).reshape(NW, PER_W)
    pe = jnp.concatenate([pe1.reshape(V1 * H), pe2.reshape(V2 * H)])
    mesh = plsc.VectorSubcoreMesh(core_axis_name="c", subcore_axis_name="s")
    out = pl.kernel(
        _pe_body,
        out_type=jax.ShapeDtypeStruct((N * D,), jnp.float32),
        mesh=mesh,
        compiler_params=pltpu.CompilerParams(needs_layout_passes=False),
        scratch_types=[
            pltpu.VMEM((PER_W,), jnp.int32),
            pltpu.VMEM((PER_W,), jnp.int32),
            pltpu.VMEM((TBL,), jnp.float32),
            pltpu.VMEM((BLK,), jnp.float32),
            pltpu.VMEM((BLK,), jnp.float32),
            pltpu.VMEM((BLK,), jnp.float32),
        ] + [pltpu.SemaphoreType.DMA] * 6,
    )(x, f0, f1, pe)
    return out.reshape(B, S, D)


# speedup vs baseline: 1.4103x; 1.4103x over previous
import jax
import jax.numpy as jnp
from jax import lax
from jax.experimental import pallas as pl
from jax.experimental.pallas import tpu as pltpu
from jax.experimental.pallas import tpu_sc as plsc

B, S, D = 1024, 200, 128
H = D // 2
N = B * S
NC, NS = 2, 16
NW = NC * NS
PER_W = N // NW
STEP = 128
N_STEPS = PER_W // STEP
NBUF = 3
L = 16
V1, V2 = 48, 256
TBL = (V1 + V2) * H
BLK = STEP * D


def _pe_body(x_hbm, f0_hbm, f1_hbm, pe_hbm, out_hbm,
             fb0, fb1, pe_v, buf0, buf1, buf2,
             six0, six1, six2, so0, so1, so2):
    bufs = (buf0, buf1, buf2)
    six = (six0, six1, six2)
    so = (so0, so1, so2)

    wid = lax.axis_index("s") * NC + lax.axis_index("c")
    base = wid * PER_W

    pltpu.sync_copy(f0_hbm.at[wid], fb0)
    pltpu.sync_copy(f1_hbm.at[wid], fb1)
    pltpu.sync_copy(pe_hbm, pe_v)

    lane = lax.iota(jnp.int32, L)
    lane_j = [jnp.int32(j * L) + lane for j in range(H // L)]
    _gdims = lax.GatherDimensionNumbers(
        offset_dims=(), collapsed_slice_dims=(0,), start_index_map=(0,))

    def bcast(vec, l):
        sel = jnp.full((L, 1), l, jnp.int32)
        return lax.gather(vec, sel, _gdims, (1,),
                          mode=lax.GatherScatterMode.PROMISE_IN_BOUNDS)

    def issue_in(g, b):
        w0 = (base + g * STEP) * D
        pltpu.async_copy(x_hbm.at[pl.ds(w0, BLK)], bufs[b], six[b])

    def wait_in(b):
        pltpu.make_async_copy(x_hbm.at[pl.ds(0, BLK)], bufs[b], six[b]).wait()

    def issue_out(g, b):
        w0 = (base + g * STEP) * D
        pltpu.async_copy(bufs[b], out_hbm.at[pl.ds(w0, BLK)], so[b])

    def wait_out(b):
        pltpu.make_async_copy(bufs[b], out_hbm.at[pl.ds(0, BLK)], so[b]).wait()

    def compute(g, b):
        buf = bufs[b]
        grow = g * STEP

        @plsc.parallel_loop(0, STEP // L, 1, unroll=2)
        def group(tg):
            t0 = tg * L
            bases0 = fb0[pl.ds(grow + t0, L)]
            bases1 = fb1[pl.ds(grow + t0, L)]
            tflat = t0 * D
            for l in range(L):
                b0 = bcast(bases0, l)
                b1 = bcast(bases1, l)
                tb = tflat + (l * D)
                vals = ([plsc.load_gather(pe_v, [b0 + lane_j[j]])
                         for j in range(H // L)] +
                        [plsc.load_gather(pe_v, [b1 + lane_j[j]])
                         for j in range(H // L)])
                for j in range(D // L):
                    plsc.addupdate(buf.at[pl.ds(tb + j * L, L)], vals[j])

    issue_in(0, 0)

    def stage_b(g, c):
        def mk(b):
            def f(_):
                nb = (b + 1) % NBUF

                def issue_next(_):
                    @pl.when(g >= NBUF - 1)
                    def _():
                        wait_out(nb)
                    issue_in(g + 1, nb)
                    return 0

                lax.cond(g + 1 < N_STEPS, issue_next, lambda _: 0, 0)
                wait_in(b)
                compute(g, b)
                issue_out(g, b)
                return 0
            return f

        lax.switch(lax.rem(g, NBUF), [mk(0), mk(1), mk(2)], 0)
        return c

    lax.fori_loop(0, N_STEPS, stage_b, 0)
    for g in range(N_STEPS - NBUF, N_STEPS):
        wait_out(g % NBUF)


@jax.jit
def kernel(inputs, pos, pe1, pe2):
    x = inputs.reshape(N * D)
    p = pos.astype(jnp.int32)
    f0 = (p[..., 0] * H).reshape(NW, PER_W)
    f1 = (p[..., 1] * H + V1 * H).reshape(NW, PER_W)
    pe = jnp.concatenate([pe1.reshape(V1 * H), pe2.reshape(V2 * H)])
    mesh = plsc.VectorSubcoreMesh(core_axis_name="c", subcore_axis_name="s")
    out = pl.kernel(
        _pe_body,
        out_type=jax.ShapeDtypeStruct((N * D,), jnp.float32),
        mesh=mesh,
        compiler_params=pltpu.CompilerParams(needs_layout_passes=False),
        scratch_types=[
            pltpu.VMEM((PER_W,), jnp.int32),
            pltpu.VMEM((PER_W,), jnp.int32),
            pltpu.VMEM((TBL,), jnp.float32),
            pltpu.VMEM((BLK,), jnp.float32),
            pltpu.VMEM((BLK,), jnp.float32),
            pltpu.VMEM((BLK,), jnp.float32),
        ] + [pltpu.SemaphoreType.DMA] * 6,
    )(x, f0, f1, pe)
    return out.reshape(B, S, D)
